# Initial kernel scaffold; baseline (speedup 1.0000x reference)
import jax
import jax.numpy as jnp
from jax import lax
from jax.experimental import pallas as pl
from jax.experimental.pallas import tpu as pltpu
from jax.experimental.pallas import tpu_sc as plsc

D = 64
NC, NS = 2, 16
NW = NC * NS
K = 128


def _sc_body(x0, x1, x2, t0, t1, t2, out, idx0, idx1, idx2, r0, r1, r2,
             s0, s1, s2):
    wid = lax.axis_index("s") * NC + lax.axis_index("c")
    tpw = out.shape[0] // NW
    nchunk = tpw // K

    def chunk_body(ci, carry):
        base = wid * tpw + ci * K
        pltpu.sync_copy(x0.at[pl.ds(base, K)], idx0)
        pltpu.sync_copy(x1.at[pl.ds(base, K)], idx1)
        pltpu.sync_copy(x2.at[pl.ds(base, K)], idx2)
        c0 = pltpu.async_copy(t0.at[idx0], r0, s0)
        c1 = pltpu.async_copy(t1.at[idx1], r1, s1)
        c2 = pltpu.async_copy(t2.at[idx2], r2, s2)
        c0.wait()
        c1.wait()
        c2.wait()

        def add_body(i, acc):
            for c in range(D // 16):
                sl = pl.ds(c * 16, 16)
                r0[i, sl] = r0[i, sl] + r1[i, sl] + r2[i, sl]
            return acc

        lax.fori_loop(0, K, add_body, 0)
        pltpu.sync_copy(r0, out.at[pl.ds(base, K)])
        return carry

    lax.fori_loop(0, nchunk, chunk_body, 0)


def kernel(x, T0, T1, T2):
    B, L, _ = x.shape
    N = B * L
    xi = x.reshape(N, 3).astype(jnp.int32)
    x0, x1, x2 = xi[:, 0], xi[:, 1], xi[:, 2]
    mesh = plsc.VectorSubcoreMesh(core_axis_name="c", subcore_axis_name="s",
                                  num_cores=NC, num_subcores=NS)
    out = pl.kernel(
        _sc_body,
        out_type=jax.ShapeDtypeStruct((N, D), jnp.float32),
        mesh=mesh,
        scratch_types=[
            pltpu.VMEM((K,), jnp.int32),
            pltpu.VMEM((K,), jnp.int32),
            pltpu.VMEM((K,), jnp.int32),
            pltpu.VMEM((K, D), jnp.float32),
            pltpu.VMEM((K, D), jnp.float32),
            pltpu.VMEM((K, D), jnp.float32),
            pltpu.SemaphoreType.DMA,
            pltpu.SemaphoreType.DMA,
            pltpu.SemaphoreType.DMA,
        ],
    )(x0, x1, x2, T0, T1, T2)
    return out.reshape(B, L, D)


# speedup vs baseline: 3.0891x; 3.0891x over previous
import jax
import jax.numpy as jnp
from jax import lax
from jax.experimental import pallas as pl
from jax.experimental.pallas import tpu as pltpu
from jax.experimental.pallas import tpu_sc as plsc

D = 64
NC, NS = 2, 16
NW = NC * NS
K = 128


def _sc_body(x0, x1, x2, t0, t1, t2, out, idx0, idx1, idx2, r0, r1, r2,
             s0, s1, s2):
    wid = lax.axis_index("s") * NC + lax.axis_index("c")
    tpw = out.shape[0] // NW
    nchunk = tpw // K

    def chunk_body(ci, carry):
        base = wid * tpw + ci * K
        pltpu.sync_copy(x0.at[pl.ds(base, K)], idx0)
        pltpu.sync_copy(x1.at[pl.ds(base, K)], idx1)
        pltpu.sync_copy(x2.at[pl.ds(base, K)], idx2)
        c0 = pltpu.async_copy(t0.at[idx0], r0, s0)
        c1 = pltpu.async_copy(t1.at[idx1], r1, s1)
        c2 = pltpu.async_copy(t2.at[idx2], r2, s2)
        c0.wait()
        c1.wait()
        c2.wait()

        def add_body(i, acc):
            for c in range(D // 16):
                sl = pl.ds(c * 16, 16)
                r0[i, sl] = r0[i, sl] + r1[i, sl] + r2[i, sl]
            return acc

        lax.fori_loop(0, K, add_body, 0)
        pltpu.sync_copy(r0, out.at[pl.ds(base, K)])
        return carry

    lax.fori_loop(0, nchunk, chunk_body, 0)


def kernel(x, T0, T1, T2):
    B, L, _ = x.shape
    N = B * L
    xi = x.reshape(N, 3).astype(jnp.int32)
    x0, x1, x2 = xi[:, 0], xi[:, 1], xi[:, 2]
    mesh = plsc.VectorSubcoreMesh(core_axis_name="c", subcore_axis_name="s",
                                  num_cores=NC, num_subcores=NS)
    out = pl.kernel(
        _sc_body,
        out_type=jax.ShapeDtypeStruct((N, D), jnp.float32),
        mesh=mesh,
        compiler_params=pltpu.CompilerParams(use_tc_tiling_on_sc=False),
        scratch_types=[
            pltpu.VMEM((K,), jnp.int32),
            pltpu.VMEM((K,), jnp.int32),
            pltpu.VMEM((K,), jnp.int32),
            pltpu.VMEM((K, D), jnp.float32),
            pltpu.VMEM((K, D), jnp.float32),
            pltpu.VMEM((K, D), jnp.float32),
            pltpu.SemaphoreType.DMA,
            pltpu.SemaphoreType.DMA,
            pltpu.SemaphoreType.DMA,
        ],
    )(x0, x1, x2, T0, T1, T2)
    return out.reshape(B, L, D)
